# Initial kernel scaffold; baseline (speedup 1.0000x reference)
import functools

import jax
import jax.numpy as jnp
from jax import lax
from jax.experimental import pallas as pl
from jax.experimental.pallas import tpu as pltpu
from jax.experimental.pallas import tpu_sc as plsc

LANES = 16
CHUNK = 128


@functools.lru_cache(maxsize=None)
def _make_embed(total_rows: int, hidden: int, pos_period: int):
    info = plsc.get_sparse_core_info()
    nc, ns = info.num_cores, info.num_subcores
    nw = nc * ns
    assert total_rows % (nw * CHUNK) == 0
    assert pos_period % CHUNK == 0
    assert hidden % LANES == 0
    rows_per_w = total_rows // nw
    nchunks = rows_per_w // CHUNK
    vregs_per_row = hidden // LANES

    mesh = plsc.VectorSubcoreMesh(core_axis_name="c", subcore_axis_name="s")

    @functools.partial(
        pl.kernel,
        out_type=jax.ShapeDtypeStruct((total_rows, hidden), jnp.float32),
        mesh=mesh,
        scratch_types=[
            pltpu.VMEM((CHUNK,), jnp.int32),
            pltpu.VMEM((CHUNK, hidden), jnp.float32),
            pltpu.VMEM((pos_period, hidden), jnp.float32),
            pltpu.SemaphoreType.DMA,
        ],
    )
    def embed(idx_hbm, table_hbm, pos_hbm, out_hbm, idx_v, rows_v, pos_v, sem):
        wid = lax.axis_index("s") * nc + lax.axis_index("c")
        base_w = wid * rows_per_w
        pltpu.sync_copy(pos_hbm, pos_v)

        def chunk_body(g, carry):
            base = base_w + g * CHUNK
            pltpu.sync_copy(idx_hbm.at[pl.ds(base, CHUNK)], idx_v)
            pltpu.async_copy(table_hbm.at[idx_v], rows_v, sem).wait()
            pos_base = lax.rem(base, pos_period)

            def row_body(i, c2):
                pr = pos_base + i
                for j in range(vregs_per_row):
                    sl = pl.ds(j * LANES, LANES)
                    rows_v[i, sl] = rows_v[i, sl] + pos_v[pr, sl]
                return c2

            lax.fori_loop(0, CHUNK, row_body, 0, unroll=2)
            pltpu.sync_copy(rows_v, out_hbm.at[pl.ds(base, CHUNK)])
            return carry

        lax.fori_loop(0, nchunks, chunk_body, 0)

    return embed


def kernel(input_grid, tok_table, pos_embed):
    b, h, w = input_grid.shape
    hidden = tok_table.shape[1]
    idx_flat = input_grid.reshape(-1)
    pos_flat = pos_embed[0, :h, :w, :].reshape(h * w, hidden)
    embed = _make_embed(b * h * w, hidden, h * w)
    out = embed(idx_flat, tok_table, pos_flat)
    return out.reshape(b, h, w, hidden)


# speedup vs baseline: 2.1609x; 2.1609x over previous
import functools

import jax
import jax.numpy as jnp
from jax import lax
from jax.experimental import pallas as pl
from jax.experimental.pallas import tpu as pltpu
from jax.experimental.pallas import tpu_sc as plsc

LANES = 16
CHUNK = 128


@functools.lru_cache(maxsize=None)
def _make_embed(total_rows: int, hidden: int, pos_period: int):
    info = plsc.get_sparse_core_info()
    nc, ns = info.num_cores, info.num_subcores
    nw = nc * ns
    assert total_rows % (nw * CHUNK) == 0
    assert pos_period % CHUNK == 0
    assert hidden % LANES == 0
    rows_per_w = total_rows // nw
    nchunks = rows_per_w // CHUNK
    vregs_per_row = hidden // LANES

    mesh = plsc.VectorSubcoreMesh(core_axis_name="c", subcore_axis_name="s")

    @functools.partial(
        pl.kernel,
        out_type=jax.ShapeDtypeStruct((total_rows, hidden), jnp.float32),
        mesh=mesh,
        scratch_types=[
            pltpu.VMEM((CHUNK,), jnp.int32),
            pltpu.VMEM((CHUNK, hidden), jnp.float32),
            pltpu.VMEM((pos_period, hidden), jnp.float32),
            pltpu.SemaphoreType.DMA,
        ],
        compiler_params=pltpu.CompilerParams(use_tc_tiling_on_sc=False),
    )
    def embed(idx_hbm, table_hbm, pos_hbm, out_hbm, idx_v, rows_v, pos_v, sem):
        wid = lax.axis_index("s") * nc + lax.axis_index("c")
        base_w = wid * rows_per_w
        pltpu.sync_copy(pos_hbm, pos_v)

        def chunk_body(g, carry):
            base = base_w + g * CHUNK
            pltpu.sync_copy(idx_hbm.at[pl.ds(base, CHUNK)], idx_v)
            pltpu.async_copy(table_hbm.at[idx_v], rows_v, sem).wait()
            pos_base = lax.rem(base, pos_period)

            def row_body(i, c2):
                pr = pos_base + i
                for j in range(vregs_per_row):
                    sl = pl.ds(j * LANES, LANES)
                    rows_v[i, sl] = rows_v[i, sl] + pos_v[pr, sl]
                return c2

            lax.fori_loop(0, CHUNK, row_body, 0, unroll=2)
            pltpu.sync_copy(rows_v, out_hbm.at[pl.ds(base, CHUNK)])
            return carry

        lax.fori_loop(0, nchunks, chunk_body, 0)

    return embed


def kernel(input_grid, tok_table, pos_embed):
    b, h, w = input_grid.shape
    hidden = tok_table.shape[1]
    idx_flat = input_grid.reshape(-1)
    pos_flat = pos_embed[0, :h, :w, :].reshape(h * w, hidden)
    embed = _make_embed(b * h * w, hidden, h * w)
    out = embed(idx_flat, tok_table, pos_flat)
    return out.reshape(b, h, w, hidden)


# speedup vs baseline: 3.1011x; 1.4351x over previous
import functools

import jax
import jax.numpy as jnp
from jax import lax
from jax.experimental import pallas as pl
from jax.experimental.pallas import tpu as pltpu
from jax.experimental.pallas import tpu_sc as plsc

LANES = 16
CHUNK = 128
NBUF = 4


@functools.lru_cache(maxsize=None)
def _make_embed(total_rows: int, hidden: int, pos_period: int):
    info = plsc.get_sparse_core_info()
    nc, ns = info.num_cores, info.num_subcores
    nw = nc * ns
    assert total_rows % (nw * CHUNK) == 0
    assert pos_period % CHUNK == 0
    assert hidden % LANES == 0
    rows_per_w = total_rows // nw
    nchunks = rows_per_w // CHUNK
    assert nchunks >= 2 * NBUF
    vregs_per_row = hidden // LANES
    chunks_per_period = pos_period // CHUNK

    mesh = plsc.VectorSubcoreMesh(core_axis_name="c", subcore_axis_name="s")

    @functools.partial(
        pl.kernel,
        out_type=jax.ShapeDtypeStruct((total_rows, hidden), jnp.float32),
        mesh=mesh,
        scratch_types=[
            pltpu.VMEM((NBUF, CHUNK), jnp.int32),
            pltpu.VMEM((NBUF, CHUNK, hidden), jnp.float32),
            pltpu.VMEM((pos_period, hidden), jnp.float32),
            pltpu.SemaphoreType.DMA((NBUF,)),
            pltpu.SemaphoreType.DMA((NBUF,)),
            pltpu.SemaphoreType.DMA((NBUF,)),
        ],
        compiler_params=pltpu.CompilerParams(use_tc_tiling_on_sc=False),
    )
    def embed(idx_hbm, table_hbm, pos_hbm, out_hbm,
              idx_ring, rows_ring, pos_v, idx_sem, g_sem, st_sem):
        wid = lax.axis_index("s") * nc + lax.axis_index("c")
        base_w = wid * rows_per_w

        def idx_src(c):
            return idx_hbm.at[pl.ds(base_w + c * CHUNK, CHUNK)]

        def out_dst(c):
            return out_hbm.at[pl.ds(base_w + c * CHUNK, CHUNK)]

        pltpu.sync_copy(pos_hbm, pos_v)

        for b in range(NBUF):
            pltpu.make_async_copy(idx_src(b), idx_ring.at[b], idx_sem.at[b]).start()
        for b in range(2):
            pltpu.make_async_copy(idx_src(b), idx_ring.at[b], idx_sem.at[b]).wait()
            pltpu.make_async_copy(
                table_hbm.at[idx_ring.at[b]], rows_ring.at[b], g_sem.at[b]).start()

        @pl.loop(0, nchunks, step=NBUF)
        def turn_outer(g):
            for b in range(NBUF):
                c = g + b
                b2 = (b + 2) % NBUF

                @pl.when(c + 2 < nchunks)
                def _():
                    @pl.when(c >= 2)
                    def _():
                        pltpu.make_async_copy(
                            rows_ring.at[b2], out_dst(c - 2), st_sem.at[b2]).wait()
                    pltpu.make_async_copy(
                        idx_src(c + 2), idx_ring.at[b2], idx_sem.at[b2]).wait()
                    pltpu.make_async_copy(
                        table_hbm.at[idx_ring.at[b2]], rows_ring.at[b2],
                        g_sem.at[b2]).start()

                pltpu.make_async_copy(
                    table_hbm.at[idx_ring.at[b]], rows_ring.at[b], g_sem.at[b]).wait()

                @pl.when(c + NBUF < nchunks)
                def _():
                    pltpu.make_async_copy(
                        idx_src(c + NBUF), idx_ring.at[b], idx_sem.at[b]).start()

                pos_base = lax.rem(c, chunks_per_period) * CHUNK

                @pl.loop(0, CHUNK, unroll=4)
                def row_body(i):
                    pr = pos_base + i
                    for j in range(vregs_per_row):
                        sl = pl.ds(j * LANES, LANES)
                        plsc.addupdate(rows_ring.at[b, i, sl], pos_v[pr, sl])

                pltpu.make_async_copy(rows_ring.at[b], out_dst(c), st_sem.at[b]).start()

        for b in range(NBUF):
            pltpu.make_async_copy(
                rows_ring.at[b], out_hbm.at[pl.ds(base_w, CHUNK)], st_sem.at[b]).wait()

    return embed


def kernel(input_grid, tok_table, pos_embed):
    b, h, w = input_grid.shape
    hidden = tok_table.shape[1]
    idx_flat = input_grid.reshape(-1)
    pos_flat = pos_embed[0, :h, :w, :].reshape(h * w, hidden)
    embed = _make_embed(b * h * w, hidden, h * w)
    out = embed(idx_flat, tok_table, pos_flat)
    return out.reshape(b, h, w, hidden)
